# Initial kernel scaffold; baseline (speedup 1.0000x reference)
import jax
import jax.numpy as jnp
from jax.experimental import pallas as pl


def kernel(idx, targets, table):
    raise NotImplementedError("write your pallas kernel here")


# speedup vs baseline: 1.2179x; 1.2179x over previous
import functools

import jax
import jax.numpy as jnp
from jax import lax
from jax.experimental import pallas as pl
from jax.experimental.pallas import tpu as pltpu
from jax.experimental.pallas import tpu_sc as plsc

VOCAB = 1000
B, T = 1024, 200
N_TOK = B * T
NC, NS = 2, 16
NW = NC * NS
PER_W = N_TOK // NW
K = 32
NCHUNK = PER_W // K
LANES = 16


def _lse_body(table_ref, out_ref):
    x = table_ref[...]
    m = jnp.max(x, axis=1)
    s = jnp.sum(jnp.exp(x - m[:, None]), axis=1)
    out_ref[...] = m + jnp.log(s)


def _lse(table):
    return pl.pallas_call(
        _lse_body,
        out_shape=jax.ShapeDtypeStruct((VOCAB,), jnp.float32),
    )(table)


_MESH = plsc.VectorSubcoreMesh(core_axis_name="c", subcore_axis_name="s")


@functools.partial(
    pl.kernel,
    out_type=[
        jax.ShapeDtypeStruct((N_TOK, VOCAB), jnp.float32),
        jax.ShapeDtypeStruct((NW, LANES), jnp.float32),
    ],
    mesh=_MESH,
    compiler_params=pltpu.CompilerParams(
        needs_layout_passes=False, use_tc_tiling_on_sc=False),
    scratch_types=[
        pltpu.VMEM((K,), jnp.int32),
        pltpu.VMEM((K,), jnp.int32),
        pltpu.VMEM((K, VOCAB), jnp.float32),
        pltpu.VMEM((VOCAB,), jnp.float32),
        pltpu.VMEM((LANES,), jnp.float32),
        pltpu.SemaphoreType.DMA,
    ],
)
def _sc_gather(idx_hbm, tgt_hbm, table_hbm, lse_hbm, out_hbm, part_hbm,
               idx_v, tgt_v, rows_v, lse_v, acc_v, sem):
    wid = lax.axis_index("s") * NC + lax.axis_index("c")
    base = wid * PER_W
    pltpu.sync_copy(lse_hbm, lse_v)

    def chunk_body(c, acc):
        off = base + c * K
        pltpu.sync_copy(idx_hbm.at[pl.ds(off, K)], idx_v)
        pltpu.sync_copy(tgt_hbm.at[pl.ds(off, K)], tgt_v)
        pltpu.async_copy(table_hbm.at[idx_v], rows_v, sem).wait()
        for g in range(K // LANES):
            rid = lax.iota(jnp.int32, LANES) + (g * LANES)
            cid = tgt_v[pl.ds(g * LANES, LANES)]
            vals = plsc.load_gather(rows_v, [rid, cid])
            iv = idx_v[pl.ds(g * LANES, LANES)]
            lses = plsc.load_gather(lse_v, [iv])
            acc = acc + (lses - vals)
        pltpu.sync_copy(rows_v, out_hbm.at[pl.ds(off, K)])
        return acc

    acc = lax.fori_loop(0, NCHUNK, chunk_body, jnp.zeros((LANES,), jnp.float32))
    acc_v[...] = acc
    pltpu.sync_copy(acc_v, part_hbm.at[wid])


def kernel(idx, targets, table):
    idx_f = idx.reshape(-1).astype(jnp.int32)
    tgt_f = targets.reshape(-1).astype(jnp.int32)
    lse = _lse(table)
    logits2, parts = _sc_gather(idx_f, tgt_f, table, lse)
    loss = jnp.sum(parts) / N_TOK
    return (logits2, loss)


# speedup vs baseline: 1.4063x; 1.1547x over previous
import functools

import jax
import jax.numpy as jnp
from jax import lax
from jax.experimental import pallas as pl
from jax.experimental.pallas import tpu as pltpu
from jax.experimental.pallas import tpu_sc as plsc

VOCAB = 1000
B, T = 1024, 200
N_TOK = B * T
NC, NS = 2, 16
NW = NC * NS
PER_W = N_TOK // NW
K = 32
NCHUNK = PER_W // K
NBUF = 3
LANES = 16


def _lse_body(table_ref, out_ref):
    x = table_ref[...]
    m = jnp.max(x, axis=1)
    s = jnp.sum(jnp.exp(x - m[:, None]), axis=1)
    out_ref[...] = m + jnp.log(s)


def _lse(table):
    return pl.pallas_call(
        _lse_body,
        out_shape=jax.ShapeDtypeStruct((VOCAB,), jnp.float32),
    )(table)


_MESH = plsc.VectorSubcoreMesh(core_axis_name="c", subcore_axis_name="s")


@functools.partial(
    pl.kernel,
    out_type=[
        jax.ShapeDtypeStruct((N_TOK, VOCAB), jnp.float32),
        jax.ShapeDtypeStruct((NW, LANES), jnp.float32),
    ],
    mesh=_MESH,
    compiler_params=pltpu.CompilerParams(
        needs_layout_passes=False, use_tc_tiling_on_sc=False),
    scratch_types=[
        pltpu.VMEM((PER_W,), jnp.int32),
        pltpu.VMEM((PER_W,), jnp.int32),
        pltpu.VMEM((NBUF, K, VOCAB), jnp.float32),
        pltpu.VMEM((VOCAB,), jnp.float32),
        pltpu.VMEM((LANES,), jnp.float32),
        pltpu.SemaphoreType.DMA((NBUF,)),
        pltpu.SemaphoreType.DMA((NBUF,)),
    ],
)
def _sc_gather(idx_hbm, tgt_hbm, table_hbm, lse_hbm, out_hbm, part_hbm,
               idx_v, tgt_v, rows_v, lse_v, acc_v, gsem, ssem):
    wid = lax.axis_index("s") * NC + lax.axis_index("c")
    base = wid * PER_W
    pltpu.sync_copy(idx_hbm.at[pl.ds(base, PER_W)], idx_v)
    pltpu.sync_copy(tgt_hbm.at[pl.ds(base, PER_W)], tgt_v)
    pltpu.sync_copy(lse_hbm, lse_v)

    def fire_gather(c, b):
        pltpu.async_copy(
            table_hbm.at[idx_v.at[pl.ds(c * K, K)]], rows_v.at[b], gsem.at[b])

    def wait_gather(c, b):
        pltpu.make_async_copy(
            table_hbm.at[idx_v.at[pl.ds(c * K, K)]], rows_v.at[b],
            gsem.at[b]).wait()

    def fire_scatter(c, b):
        pltpu.async_copy(
            rows_v.at[b], out_hbm.at[pl.ds(base + c * K, K)], ssem.at[b])

    def wait_scatter(c, b):
        pltpu.make_async_copy(
            rows_v.at[b], out_hbm.at[pl.ds(base + c * K, K)],
            ssem.at[b]).wait()

    def extract(c, b, acc):
        for g in range(K // LANES):
            rid = lax.iota(jnp.int32, LANES) + (g * LANES)
            cid = tgt_v[pl.ds(c * K + g * LANES, LANES)]
            vals = plsc.load_gather(rows_v.at[b], [rid, cid])
            iv = idx_v[pl.ds(c * K + g * LANES, LANES)]
            lses = plsc.load_gather(lse_v, [iv])
            acc = acc + (lses - vals)
        return acc

    acc = jnp.zeros((LANES,), jnp.float32)

    fire_gather(0, 0)
    fire_gather(1, 1)
    wait_gather(0, 0)
    acc = extract(0, 0, acc)
    fire_scatter(0, 0)
    fire_gather(2, 2)
    wait_gather(1, 1)
    acc = extract(1, 1, acc)
    fire_scatter(1, 1)

    def step(c, acc):
        bn = (c + 1) % NBUF
        wait_scatter(c - 2, bn)
        fire_gather(c + 1, bn)
        b = c % NBUF
        wait_gather(c, b)
        acc = extract(c, b, acc)
        fire_scatter(c, b)
        return acc

    def outer(i, acc):
        c0 = 2 + i * NBUF
        for j in range(NBUF):
            acc = step(c0 + j, acc)
        return acc

    n_steady = NCHUNK - 3
    n_outer = n_steady // NBUF
    acc = lax.fori_loop(0, n_outer, outer, acc)
    for c in range(2 + n_outer * NBUF, NCHUNK - 1):
        acc = step(c, acc)

    c_last = NCHUNK - 1
    b = c_last % NBUF
    wait_gather(c_last, b)
    acc = extract(c_last, b, acc)
    fire_scatter(c_last, b)
    for c in range(NCHUNK - NBUF, NCHUNK):
        wait_scatter(c, c % NBUF)

    acc_v[...] = acc
    pltpu.sync_copy(acc_v, part_hbm.at[wid])


def kernel(idx, targets, table):
    idx_f = idx.reshape(-1).astype(jnp.int32)
    tgt_f = targets.reshape(-1).astype(jnp.int32)
    lse = _lse(table)
    logits2, parts = _sc_gather(idx_f, tgt_f, table, lse)
    loss = jnp.sum(parts) / N_TOK
    return (logits2, loss)


# speedup vs baseline: 2.1102x; 1.5005x over previous
import functools

import jax
import jax.numpy as jnp
from jax import lax
from jax.experimental import pallas as pl
from jax.experimental.pallas import tpu as pltpu
from jax.experimental.pallas import tpu_sc as plsc

VOCAB = 1000
VPAD = 1024
B, T = 1024, 200
N_TOK = B * T
NC, NS = 2, 16
NW = NC * NS
PER_W = N_TOK // NW
K = 32
NCHUNK = PER_W // K
NBUF = 3
LANES = 16
KE = 128
NECHUNK = PER_W // KE


def _lse_body(table_ref, out_ref):
    x = table_ref[...]
    m = jnp.max(x, axis=1)
    s = jnp.sum(jnp.exp(x - m[:, None]), axis=1)
    out_ref[...] = m + jnp.log(s)


def _lse(table):
    return pl.pallas_call(
        _lse_body,
        out_shape=jax.ShapeDtypeStruct((VOCAB,), jnp.float32),
    )(table)


_MESH = plsc.VectorSubcoreMesh(core_axis_name="c", subcore_axis_name="s")


VMAIN = 896
VTAIL = VOCAB - VMAIN
_WINS = [0, 16, 32, 48, 64, 80, 88]


@functools.partial(
    pl.kernel,
    out_type=jax.ShapeDtypeStruct((N_TOK, VOCAB), jnp.float32),
    mesh=_MESH,
    compiler_params=pltpu.CompilerParams(use_tc_tiling_on_sc=True),
    scratch_types=[
        pltpu.VMEM((PER_W,), jnp.int32),
        pltpu.VMEM((NBUF, K, VPAD), jnp.float32),
        pltpu.VMEM((NBUF, K, VTAIL), jnp.float32),
        pltpu.SemaphoreType.DMA((NBUF,)),
        pltpu.SemaphoreType.DMA((NBUF,)),
    ],
)
def _sc_gather(idx_hbm, tpad_hbm, out_hbm, idx_v, rows_v, tail_v, gsem, ssem):
    wid = lax.axis_index("s") * NC + lax.axis_index("c")
    base = wid * PER_W
    pltpu.sync_copy(idx_hbm.at[pl.ds(base, PER_W)], idx_v)

    def fire_gather(c, b):
        pltpu.async_copy(
            tpad_hbm.at[idx_v.at[pl.ds(c * K, K)]], rows_v.at[b], gsem.at[b])

    def wait_gather(c, b):
        pltpu.make_async_copy(
            tpad_hbm.at[idx_v.at[pl.ds(c * K, K)]], rows_v.at[b],
            gsem.at[b]).wait()

    def repack_tail(b):
        for j in range(K):
            for w in _WINS:
                tail_v[b, j, pl.ds(w, LANES)] = \
                    rows_v[b, j, pl.ds(VMAIN + w, LANES)]

    def fire_scatter(c, b):
        off = base + c * K
        pltpu.async_copy(
            rows_v.at[b, :, pl.ds(0, VMAIN)],
            out_hbm.at[pl.ds(off, K), pl.ds(0, VMAIN)], ssem.at[b])
        pltpu.async_copy(
            tail_v.at[b],
            out_hbm.at[pl.ds(off, K), pl.ds(VMAIN, VTAIL)], ssem.at[b])

    def wait_scatter(c, b):
        off = base + c * K
        pltpu.make_async_copy(
            rows_v.at[b, :, pl.ds(0, VMAIN)],
            out_hbm.at[pl.ds(off, K), pl.ds(0, VMAIN)], ssem.at[b]).wait()
        pltpu.make_async_copy(
            tail_v.at[b],
            out_hbm.at[pl.ds(off, K), pl.ds(VMAIN, VTAIL)],
            ssem.at[b]).wait()

    fire_gather(0, 0)
    fire_gather(1, 1)
    wait_gather(0, 0)
    repack_tail(0)
    fire_scatter(0, 0)
    fire_gather(2, 2)
    wait_gather(1, 1)
    repack_tail(1)
    fire_scatter(1, 1)

    def step(c, _):
        bn = (c + 1) % NBUF
        wait_scatter(c - 2, bn)
        fire_gather(c + 1, bn)
        b = c % NBUF
        wait_gather(c, b)
        repack_tail(b)
        fire_scatter(c, b)
        return 0

    def outer(i, carry):
        c0 = 2 + i * NBUF
        for j in range(NBUF):
            carry = step(c0 + j, carry)
        return carry

    n_steady = NCHUNK - 3
    n_outer = n_steady // NBUF
    carry = lax.fori_loop(0, n_outer, outer, 0)
    for c in range(2 + n_outer * NBUF, NCHUNK - 1):
        carry = step(c, carry)

    c_last = NCHUNK - 1
    wait_gather(c_last, c_last % NBUF)
    repack_tail(c_last % NBUF)
    fire_scatter(c_last, c_last % NBUF)
    for c in range(NCHUNK - NBUF, NCHUNK):
        wait_scatter(c, c % NBUF)


@functools.partial(
    pl.kernel,
    out_type=jax.ShapeDtypeStruct((NW * LANES,), jnp.float32),
    mesh=_MESH,
    compiler_params=pltpu.CompilerParams(
        needs_layout_passes=False, use_tc_tiling_on_sc=False),
    scratch_types=[
        pltpu.VMEM((PER_W,), jnp.int32),
        pltpu.VMEM((PER_W,), jnp.int32),
        pltpu.VMEM((2, KE), jnp.int32),
        pltpu.VMEM((2, KE), jnp.float32),
        pltpu.VMEM((VOCAB,), jnp.float32),
        pltpu.VMEM((LANES,), jnp.float32),
        pltpu.SemaphoreType.DMA((2,)),
    ],
)
def _sc_loss(idx_hbm, tgt_hbm, tflat_hbm, lse_hbm, part_hbm,
             idx_v, tgt_v, fid_v, val_v, lse_v, acc_v, vsem):
    wid = lax.axis_index("s") * NC + lax.axis_index("c")
    base = wid * PER_W
    pltpu.sync_copy(idx_hbm.at[pl.ds(base, PER_W)], idx_v)
    pltpu.sync_copy(tgt_hbm.at[pl.ds(base, PER_W)], tgt_v)
    pltpu.sync_copy(lse_hbm, lse_v)

    def build_fids(c, b):
        for g in range(KE // LANES):
            o = c * KE + g * LANES
            iv = idx_v[pl.ds(o, LANES)]
            cv = tgt_v[pl.ds(o, LANES)]
            fid_v[b, pl.ds(g * LANES, LANES)] = iv * VOCAB + cv

    def fire(c, b):
        pltpu.async_copy(
            tflat_hbm.at[fid_v.at[b]], val_v.at[b], vsem.at[b])

    def wait(c, b):
        pltpu.make_async_copy(
            tflat_hbm.at[fid_v.at[b]], val_v.at[b], vsem.at[b]).wait()

    def consume(c, b, acc):
        for g in range(KE // LANES):
            o = c * KE + g * LANES
            iv = idx_v[pl.ds(o, LANES)]
            lses = plsc.load_gather(lse_v, [iv])
            acc = acc + (lses - val_v[b, pl.ds(g * LANES, LANES)])
        return acc

    build_fids(0, 0)
    fire(0, 0)

    def step(c, acc):
        b = c % 2
        bn = 1 - b
        build_fids(c + 1, bn)
        fire(c + 1, bn)
        wait(c, b)
        return consume(c, b, acc)

    def outer(i, acc):
        c0 = i * 2
        acc = step(c0, acc)
        acc = step(c0 + 1, acc)
        return acc

    acc = jnp.zeros((LANES,), jnp.float32)
    n_steady = NECHUNK - 1
    n_outer = n_steady // 2
    acc = lax.fori_loop(0, n_outer, outer, acc)
    for c in range(n_outer * 2, NECHUNK - 1):
        acc = step(c, acc)
    c_last = NECHUNK - 1
    wait(c_last, c_last % 2)
    acc = consume(c_last, c_last % 2, acc)

    acc_v[...] = acc
    pltpu.sync_copy(acc_v, part_hbm.at[pl.ds(wid * LANES, LANES)])


def kernel(idx, targets, table):
    idx_f = idx.reshape(-1).astype(jnp.int32)
    tgt_f = targets.reshape(-1).astype(jnp.int32)
    tpad = jnp.pad(table, ((0, 0), (0, VPAD - VOCAB)))
    lse = _lse(table)
    logits2 = _sc_gather(idx_f, tpad)
    parts = _sc_loss(idx_f, tgt_f, table.reshape(-1), lse)
    loss = jnp.sum(parts) / N_TOK
    return (logits2, loss)
